# baseline (speedup 1.0000x reference)
import functools
import jax
import jax.numpy as jnp
from jax import lax
from jax.experimental import pallas as pl
from jax.experimental.pallas import tpu as pltpu, tpu_sc as plsc

SEQ = 2048
HID = 1024

_NC, _NS = 2, 16
_NW = _NC * _NS
_ROWS = SEQ // _NW


def _make_sc_copy():
    mesh = plsc.VectorSubcoreMesh(
        core_axis_name="c", subcore_axis_name="s",
        num_cores=_NC, num_subcores=_NS,
    )

    @functools.partial(
        pl.kernel,
        mesh=mesh,
        out_type=jax.ShapeDtypeStruct((SEQ, HID), jnp.float32),
        scratch_types=[
            pltpu.VMEM((_ROWS, HID), jnp.float32),
            pltpu.SemaphoreType.DMA,
        ],
    )
    def sc_copy(table_hbm, out_hbm, buf, sem):
        del buf, sem
        wid = lax.axis_index("s") * _NC + lax.axis_index("c")
        base = wid * _ROWS
        pltpu.sync_copy(table_hbm.at[pl.ds(base, _ROWS)],
                        out_hbm.at[pl.ds(base, _ROWS)])

    return sc_copy


_sc_copy = _make_sc_copy()


def kernel(input_ids, table):
    del input_ids
    return _sc_copy(table)[None]


# speedup vs baseline: 10.9249x; 10.9249x over previous
import functools
import jax
import jax.numpy as jnp
from jax import lax
from jax.experimental import pallas as pl
from jax.experimental.pallas import tpu as pltpu, tpu_sc as plsc

SEQ = 2048
HID = 1024

_NC, _NS = 2, 16
_NW = _NC * _NS
_ROWS = SEQ // _NW


def _make_sc_copy():
    mesh = plsc.VectorSubcoreMesh(
        core_axis_name="c", subcore_axis_name="s",
        num_cores=_NC, num_subcores=_NS,
    )

    @functools.partial(
        pl.kernel,
        mesh=mesh,
        out_type=jax.ShapeDtypeStruct((SEQ, HID), jnp.float32),
        scratch_types=[
            pltpu.VMEM((_ROWS, HID), jnp.float32),
            pltpu.SemaphoreType.DMA,
        ],
    )
    def sc_copy(table_hbm, out_hbm, buf, sem):
        wid = lax.axis_index("s") * _NC + lax.axis_index("c")
        base = wid * _ROWS
        pltpu.sync_copy(table_hbm.at[pl.ds(base, _ROWS)], buf)
        pltpu.sync_copy(buf, out_hbm.at[pl.ds(base, _ROWS)])

    return sc_copy


_sc_copy = _make_sc_copy()


def kernel(input_ids, table):
    del input_ids
    return _sc_copy(table)[None]
